# baseline (speedup 1.0000x reference)
import functools

import jax
import jax.numpy as jnp
from jax import lax
from jax.experimental import pallas as pl
from jax.experimental.pallas import tpu as pltpu
from jax.experimental.pallas import tpu_sc as plsc

A, BS, T, V = 8, 4096, 50, 32
M = A * BS * T
NC, NS = 2, 16
NW = NC * NS
SUBR = 1024
SR_PER_W = 50
PER_W = SUBR * SR_PER_W
SR_PER_CH = 5
CH = SUBR * SR_PER_CH
N_CHUNKS = SR_PER_W // SR_PER_CH
JV = SUBR // 16

_mesh = plsc.VectorSubcoreMesh(core_axis_name="c", subcore_axis_name="s")


@functools.partial(
    pl.kernel,
    mesh=_mesh,
    out_type=jax.ShapeDtypeStruct((NW, SR_PER_W * 16), jnp.float32),
    compiler_params=pltpu.CompilerParams(needs_layout_passes=False),
    scratch_types=[
        pltpu.VMEM((CH,), jnp.int32),
        pltpu.VMEM((CH,), jnp.int32),
        pltpu.VMEM((CH,), jnp.float32),
        pltpu.VMEM((CH,), jnp.float32),
        pltpu.VMEM((CH,), jnp.int32),
        pltpu.VMEM((CH,), jnp.int32),
        pltpu.VMEM((CH,), jnp.float32),
        pltpu.VMEM((CH,), jnp.float32),
        pltpu.VMEM((SR_PER_W * 16,), jnp.float32),
        pltpu.SemaphoreType.DMA,
        pltpu.SemaphoreType.DMA,
        pltpu.SemaphoreType.DMA,
    ],
)
def _sc_gather_reduce(lp_hbm, act_hbm, td_hbm, out_hbm,
                      act0, act1, td0, td1, idx0, idx1, gat0, gat1,
                      acc_v, sem_in, sem_g0, sem_g1):
    c = lax.axis_index("c")
    s = lax.axis_index("s")
    wid = s * NC + c
    pbase = wid * PER_W
    g0 = wid * SR_PER_W
    lanes = lax.iota(jnp.int32, 16)
    act_b, td_b, idx_b, gat_b = (act0, act1), (td0, td1), (idx0, idx1), (gat0, gat1)
    sem_g = (sem_g0, sem_g1)

    def load_and_index(ci, b):
        cb = pbase + ci * CH
        pltpu.async_copy(act_hbm.at[pl.ds(cb, CH)], act_b[b], sem_in)
        pltpu.async_copy(td_hbm.at[pl.ds(cb, CH)], td_b[b], sem_in).wait()
        pltpu.make_async_copy(act_hbm.at[pl.ds(cb, CH)], act_b[b],
                              sem_in).wait()

        def sub_body(r, _):
            g = g0 + ci * SR_PER_CH + r
            plane = (g >> 2) * (V * BS)
            b0 = (g & 3) << 10

            def idx_body(j, _):
                bj = b0 + j * 16
                sb = plane + ((bj >> 7) << 10) + (bj & 127)
                av = act_b[b][pl.ds(r * SUBR + j * 16, 16)]
                idx_b[b][pl.ds(r * SUBR + j * 16, 16)] = (
                    (sb + lanes) + ((av >> 3) << 15) + ((av & 7) << 7))
                return 0
            lax.fori_loop(0, JV, idx_body, 0, unroll=8)
            return 0
        lax.fori_loop(0, SR_PER_CH, sub_body, 0)

    def fire(b):
        pltpu.async_copy(lp_hbm.at[idx_b[b]], gat_b[b], sem_g[b])

    def drain(b):
        pltpu.make_async_copy(lp_hbm.at[idx_b[b]], gat_b[b], sem_g[b]).wait()

    def accumulate(ci, b):
        def sub_body(r, _):
            def acc_body(j, av):
                d = r * SUBR + j * 16
                return av + gat_b[b][pl.ds(d, 16)] * td_b[b][pl.ds(d, 16)]
            av = lax.fori_loop(0, JV, acc_body, jnp.zeros((16,), jnp.float32),
                               unroll=8)
            acc_v[pl.ds((ci * SR_PER_CH + r) * 16, 16)] = av
            return 0
        lax.fori_loop(0, SR_PER_CH, sub_body, 0)

    load_and_index(0, 0)
    fire(0)
    load_and_index(1, 1)
    fire(1)

    def step(cc, b):
        drain(b)
        accumulate(cc, b)
        load_and_index(cc + 2, b)
        fire(b)

    def pair(i, _):
        step(2 * i, 0)
        step(2 * i + 1, 1)
        return 0

    lax.fori_loop(0, N_CHUNKS // 2 - 1, pair, 0)
    drain(0)
    accumulate(N_CHUNKS - 2, 0)
    drain(1)
    accumulate(N_CHUNKS - 1, 1)
    pltpu.sync_copy(acc_v, out_hbm.at[wid])


def kernel(log_policies, td_errors, actions):
    lp_flat = jnp.transpose(
        jnp.transpose(log_policies, (0, 2, 3, 1))
        .reshape(A, T, V // 8, 8, BS // 128, 128),
        (0, 1, 2, 4, 3, 5)).reshape(-1)
    act_flat = jnp.transpose(actions.astype(jnp.int32), (0, 2, 3, 1)).reshape(-1)
    td_flat = jnp.transpose(td_errors.astype(jnp.float32), (0, 2, 3, 1)).reshape(-1)
    partials = _sc_gather_reduce(lp_flat, act_flat, td_flat)
    sums = partials.reshape(A, T, 4 * 16).sum(axis=-1)
    return sums * (-1.0 / BS)
